# baseline (speedup 1.0000x reference)
import functools

import jax
import jax.numpy as jnp
from jax import lax
from jax.experimental import pallas as pl
from jax.experimental.pallas import tpu as pltpu
from jax.experimental.pallas import tpu_sc as plsc

N = 10000
D = 128
NPAD = 10240
NTILES = 32
BLK = 128
NB_F = 112
NB_S = 46
EPAD = 16 * (NB_F + NB_S) * BLK
ROWS_PT = NPAD // 16

_mesh = plsc.VectorSubcoreMesh(core_axis_name="c", subcore_axis_name="s")


@functools.partial(
    pl.kernel,
    out_type=jax.ShapeDtypeStruct((2, NPAD), jnp.float32),
    mesh=_mesh,
    scratch_types=[
        pltpu.VMEM((NB_F, BLK), jnp.int32),
        pltpu.VMEM((BLK,), jnp.float32),
        pltpu.VMEM((ROWS_PT,), jnp.float32),
        pltpu.VMEM_SHARED((NPAD,), jnp.float32),
    ],
)
def _deg_kernel(dst_f, dst_s, out_hbm, dst_v, ones_v, zb_v, deg_sp):
    cid = lax.axis_index("c")
    sid = lax.axis_index("s")

    def fill_zeros(i, _):
        zb_v[pl.ds(i * 16, 16)] = jnp.zeros((16,), jnp.float32)
        return 0
    lax.fori_loop(0, ROWS_PT // 16, fill_zeros, 0)

    def fill_ones(i, _):
        ones_v[pl.ds(i * 16, 16)] = jnp.ones((16,), jnp.float32)
        return 0
    lax.fori_loop(0, BLK // 16, fill_ones, 0)

    pltpu.sync_copy(zb_v, deg_sp.at[pl.ds(sid * ROWS_PT, ROWS_PT)])

    def scan(dst_hbm, nb):
        pltpu.sync_copy(dst_hbm.at[sid], dst_v.at[pl.ds(0, nb)])
        plsc.subcore_barrier()

        def body(j, _):
            pltpu.sync_copy(ones_v, deg_sp.at[dst_v.at[j]], add=True)
            return 0
        lax.fori_loop(0, nb, body, 0)

    @pl.when(cid == 0)
    def _():
        scan(dst_f, NB_F)

    @pl.when(cid == 1)
    def _():
        scan(dst_s, NB_S)

    plsc.subcore_barrier()
    pltpu.sync_copy(
        deg_sp.at[pl.ds(sid * ROWS_PT, ROWS_PT)],
        out_hbm.at[cid, pl.ds(sid * ROWS_PT, ROWS_PT)],
    )


@functools.partial(
    pl.kernel,
    out_type=jax.ShapeDtypeStruct((2, NPAD, D), jnp.float32),
    mesh=_mesh,
    scratch_types=[
        pltpu.VMEM((NB_F // 2, BLK), jnp.int32),
        pltpu.VMEM((NB_F // 2, BLK), jnp.int32),
        pltpu.VMEM((2, BLK, D), jnp.float32),
        pltpu.VMEM_SHARED((NPAD, D), jnp.float32),
        pltpu.SemaphoreType.DMA, pltpu.SemaphoreType.DMA,
    ],
)
def _prop_kernel(y_hbm, src_f, dst_f, src_s, dst_s, out_hbm,
                 src_v, dst_v, rows_v, p_sp, g0, g1):
    cid = lax.axis_index("c")
    sid = lax.axis_index("s")
    gs = (g0, g1)

    def fill_zeros(i, _):
        for l in range(D // 16):
            rows_v[0, i, pl.ds(l * 16, 16)] = jnp.zeros((16,), jnp.float32)
        return 0
    lax.fori_loop(0, BLK, fill_zeros, 0)

    def zcp(k, _):
        pltpu.sync_copy(rows_v.at[0],
                        p_sp.at[pl.ds(sid * ROWS_PT + k * BLK, BLK)])
        return 0
    lax.fori_loop(0, ROWS_PT // BLK, zcp, 0)
    plsc.subcore_barrier()

    @pl.when(cid == 0)
    def _():
        nh = NB_F // 2
        for half in range(2):
            pltpu.sync_copy(src_f.at[sid, pl.ds(half * nh, nh)], src_v)
            pltpu.sync_copy(dst_f.at[sid, pl.ds(half * nh, nh)], dst_v)

            def body(l, _):
                pltpu.async_copy(y_hbm.at[src_v.at[l]], rows_v.at[0], g0).wait()
                pltpu.sync_copy(rows_v.at[0], p_sp.at[dst_v.at[l]], add=True)
                return 0
            lax.fori_loop(0, nh, body, 0)

    @pl.when(cid == 1)
    def _():
        pltpu.sync_copy(src_s.at[sid], src_v.at[pl.ds(0, NB_S)])
        pltpu.sync_copy(dst_s.at[sid], dst_v.at[pl.ds(0, NB_S)])
        pltpu.async_copy(y_hbm.at[src_v.at[0]], rows_v.at[0], gs[0])

        def step(l, b, prefetch):
            pltpu.make_async_copy(
                y_hbm.at[src_v.at[l]], rows_v.at[b], gs[b]).wait()
            if prefetch:
                pltpu.async_copy(
                    y_hbm.at[src_v.at[l + 1]], rows_v.at[1 - b], gs[1 - b])
            pltpu.sync_copy(rows_v.at[b], p_sp.at[dst_v.at[l]], add=True)

        def body(m, _):
            step(2 * m, 0, True)
            step(2 * m + 1, 1, True)
            return 0
        lax.fori_loop(0, NB_S // 2 - 1, body, 0)
        step(NB_S - 2, 0, True)
        step(NB_S - 1, 1, False)

    plsc.subcore_barrier()
    pltpu.sync_copy(
        p_sp.at[pl.ds(sid * ROWS_PT, ROWS_PT)],
        out_hbm.at[cid, pl.ds(sid * ROWS_PT, ROWS_PT)],
    )


def _scale_body(deg_ref, x_ref, y_ref):
    deg = deg_ref[0] + deg_ref[1] + 1.0
    y_ref[...] = x_ref[...] * jax.lax.rsqrt(deg)[:, None]


def _scale(degp, xp):
    blk = 2048
    return pl.pallas_call(
        _scale_body,
        grid=(NPAD // blk,),
        in_specs=[
            pl.BlockSpec((2, blk), lambda i: (0, i)),
            pl.BlockSpec((blk, D), lambda i: (i, 0)),
        ],
        out_specs=pl.BlockSpec((blk, D), lambda i: (i, 0)),
        out_shape=jax.ShapeDtypeStruct((NPAD, D), jnp.float32),
    )(degp, xp)


def _dense_body(degp, pp, y, h, Wz, Wr, Wh, Wlz, Wlr, Wlh, Wlin,
                bz, br, bh, blz, blr, blh, blin, z_out, h_out):
    deg = degp[0] + degp[1] + 1.0
    dinv = jax.lax.rsqrt(deg)[:, None]
    A = dinv * (pp[0] + pp[1] + y[...])
    hh = h[...]

    def mm(a, b):
        return jnp.dot(a, b, preferred_element_type=jnp.float32)

    wlz = Wlz[...]
    wlr = Wlr[...]
    wlh = Wlh[...]
    convz = mm(A, Wz[...]) + bz[...]
    Z = jax.nn.sigmoid(mm(convz, wlz[:D]) + mm(hh, wlz[D:]) + blz[...])
    convr = mm(A, Wr[...]) + br[...]
    Rg = jax.nn.sigmoid(mm(convr, wlr[:D]) + mm(hh, wlr[D:]) + blr[...])
    convh = mm(A, Wh[...]) + bh[...]
    Ht = jnp.tanh(mm(convh, wlh[:D]) + mm(hh * Rg, wlh[D:]) + blh[...])
    H = Z * hh + (1.0 - Z) * Ht
    h_out[...] = H
    z_out[...] = mm(jax.nn.relu(H), Wlin[...]) + blin[...]


def _dense(degp, pparts, y, hp, W_z, W_r, W_h, Wl_z, Wl_r, Wl_h, W_lin,
           b_z, b_r, b_h, bl_z, bl_r, bl_h, b_lin):
    blk = 512
    full = lambda shape: pl.BlockSpec(shape, lambda i: tuple(0 for _ in shape))
    return pl.pallas_call(
        _dense_body,
        grid=(NPAD // blk,),
        in_specs=[
            pl.BlockSpec((2, blk), lambda i: (0, i)),
            pl.BlockSpec((2, blk, D), lambda i: (0, i, 0)),
            pl.BlockSpec((blk, D), lambda i: (i, 0)),
            pl.BlockSpec((blk, D), lambda i: (i, 0)),
            full((D, D)), full((D, D)), full((D, D)),
            full((2 * D, D)), full((2 * D, D)), full((2 * D, D)),
            full((D, D)),
            full((1, D)), full((1, D)), full((1, D)),
            full((1, D)), full((1, D)), full((1, D)), full((1, D)),
        ],
        out_specs=[
            pl.BlockSpec((blk, D), lambda i: (i, 0)),
            pl.BlockSpec((blk, D), lambda i: (i, 0)),
        ],
        out_shape=[
            jax.ShapeDtypeStruct((NPAD, D), jnp.float32),
            jax.ShapeDtypeStruct((NPAD, D), jnp.float32),
        ],
    )(degp, pparts, y, hp, W_z, W_r, W_h, Wl_z, Wl_r, Wl_h, W_lin,
      b_z.reshape(1, D), b_r.reshape(1, D), b_h.reshape(1, D),
      bl_z.reshape(1, D), bl_r.reshape(1, D), bl_h.reshape(1, D),
      b_lin.reshape(1, D))


def kernel(node_feat, src, dst, h, W_z, b_z, Wl_z, bl_z, W_r, b_r, Wl_r, bl_r,
           W_h, b_h, Wl_h, bl_h, W_lin, b_lin):
    pad_e = EPAD - src.shape[0]
    srcp = jnp.concatenate([src, jnp.full((pad_e,), N, jnp.int32)])
    dstp = jnp.concatenate([dst, jnp.full((pad_e,), N, jnp.int32)])
    ef = 16 * NB_F * BLK
    src_f = srcp[:ef].reshape(16, NB_F, BLK)
    src_s = srcp[ef:].reshape(16, NB_S, BLK)
    dst_f = dstp[:ef].reshape(16, NB_F, BLK)
    dst_s = dstp[ef:].reshape(16, NB_S, BLK)
    xp = jnp.zeros((NPAD, D), jnp.float32).at[:N].set(node_feat)
    hp = jnp.zeros((NPAD, D), jnp.float32).at[:N].set(h)

    degp = _deg_kernel(dst_f, dst_s)
    y = _scale(degp, xp)
    pparts = _prop_kernel(y, src_f, dst_f, src_s, dst_s)
    z_pad, H_pad = _dense(degp, pparts, y, hp, W_z, W_r, W_h, Wl_z, Wl_r,
                          Wl_h, W_lin, b_z, b_r, b_h, bl_z, bl_r, bl_h, b_lin)
    return (z_pad[:N], H_pad[:N])


# speedup vs baseline: 1.0926x; 1.0926x over previous
import functools

import jax
import jax.numpy as jnp
from jax import lax
from jax.experimental import pallas as pl
from jax.experimental.pallas import tpu as pltpu
from jax.experimental.pallas import tpu_sc as plsc

N = 10000
D = 128
NPAD = 10240
NTILES = 32
BLK = 128
NB_F = 112
NB_S = 46
EPAD = 16 * (NB_F + NB_S) * BLK
ROWS_PT = NPAD // 16

_mesh = plsc.VectorSubcoreMesh(core_axis_name="c", subcore_axis_name="s")


@functools.partial(
    pl.kernel,
    out_type=jax.ShapeDtypeStruct((2, NPAD), jnp.float32),
    mesh=_mesh,
    scratch_types=[
        pltpu.VMEM((NB_F, BLK), jnp.int32),
        pltpu.VMEM((BLK,), jnp.float32),
        pltpu.VMEM((ROWS_PT,), jnp.float32),
        pltpu.VMEM_SHARED((NPAD,), jnp.float32),
    ],
)
def _deg_kernel(dst_f, dst_s, out_hbm, dst_v, ones_v, zb_v, deg_sp):
    cid = lax.axis_index("c")
    sid = lax.axis_index("s")

    def fill_zeros(i, _):
        zb_v[pl.ds(i * 16, 16)] = jnp.zeros((16,), jnp.float32)
        return 0
    lax.fori_loop(0, ROWS_PT // 16, fill_zeros, 0)

    def fill_ones(i, _):
        ones_v[pl.ds(i * 16, 16)] = jnp.ones((16,), jnp.float32)
        return 0
    lax.fori_loop(0, BLK // 16, fill_ones, 0)

    pltpu.sync_copy(zb_v, deg_sp.at[pl.ds(sid * ROWS_PT, ROWS_PT)])

    def scan(dst_hbm, nb):
        pltpu.sync_copy(dst_hbm.at[sid], dst_v.at[pl.ds(0, nb)])
        plsc.subcore_barrier()

        def body(j, _):
            pltpu.sync_copy(ones_v, deg_sp.at[dst_v.at[j]], add=True)
            return 0
        lax.fori_loop(0, nb, body, 0)

    @pl.when(cid == 0)
    def _():
        scan(dst_f, NB_F)

    @pl.when(cid == 1)
    def _():
        scan(dst_s, NB_S)

    plsc.subcore_barrier()
    pltpu.sync_copy(
        deg_sp.at[pl.ds(sid * ROWS_PT, ROWS_PT)],
        out_hbm.at[cid, pl.ds(sid * ROWS_PT, ROWS_PT)],
    )


@functools.partial(
    pl.kernel,
    out_type=jax.ShapeDtypeStruct((2, NPAD, D), jnp.float32),
    mesh=_mesh,
    scratch_types=[
        pltpu.VMEM((NB_F // 2, BLK), jnp.int32),
        pltpu.VMEM((NB_F // 2, BLK), jnp.int32),
        pltpu.VMEM((2, BLK, D), jnp.float32),
        pltpu.VMEM_SHARED((NPAD, D), jnp.float32),
        pltpu.SemaphoreType.DMA, pltpu.SemaphoreType.DMA,
    ],
)
def _prop_kernel(y_hbm, src_f, dst_f, src_s, dst_s, out_hbm,
                 src_v, dst_v, rows_v, p_sp, g0, g1):
    cid = lax.axis_index("c")
    sid = lax.axis_index("s")
    gs = (g0, g1)

    def fill_zeros(i, _):
        for l in range(D // 16):
            rows_v[0, i, pl.ds(l * 16, 16)] = jnp.zeros((16,), jnp.float32)
        return 0
    lax.fori_loop(0, BLK, fill_zeros, 0)

    def zcp(k, _):
        pltpu.sync_copy(rows_v.at[0],
                        p_sp.at[pl.ds(sid * ROWS_PT + k * BLK, BLK)])
        return 0
    lax.fori_loop(0, ROWS_PT // BLK, zcp, 0)
    plsc.subcore_barrier()

    def step(l, b, prefetch):
        pltpu.make_async_copy(
            y_hbm.at[src_v.at[l]], rows_v.at[b], gs[b]).wait()
        if prefetch:
            pltpu.async_copy(
                y_hbm.at[src_v.at[l + 1]], rows_v.at[1 - b], gs[1 - b])
        pltpu.sync_copy(rows_v.at[b], p_sp.at[dst_v.at[l]], add=True)

    def run_prefetch(nb):
        pltpu.async_copy(y_hbm.at[src_v.at[0]], rows_v.at[0], gs[0])

        def body(m, _):
            step(2 * m, 0, True)
            step(2 * m + 1, 1, True)
            return 0
        lax.fori_loop(0, nb // 2 - 1, body, 0)
        step(nb - 2, 0, True)
        step(nb - 1, 1, False)

    @pl.when(cid == 0)
    def _():
        nh = NB_F // 2
        for half in range(2):
            pltpu.sync_copy(src_f.at[sid, pl.ds(half * nh, nh)], src_v)
            pltpu.sync_copy(dst_f.at[sid, pl.ds(half * nh, nh)], dst_v)
            run_prefetch(nh)

    @pl.when(cid == 1)
    def _():
        pltpu.sync_copy(src_s.at[sid], src_v.at[pl.ds(0, NB_S)])
        pltpu.sync_copy(dst_s.at[sid], dst_v.at[pl.ds(0, NB_S)])
        run_prefetch(NB_S)

    plsc.subcore_barrier()
    pltpu.sync_copy(
        p_sp.at[pl.ds(sid * ROWS_PT, ROWS_PT)],
        out_hbm.at[cid, pl.ds(sid * ROWS_PT, ROWS_PT)],
    )


def _scale_body(deg_ref, x_ref, y_ref):
    deg = deg_ref[0] + deg_ref[1] + 1.0
    y_ref[...] = x_ref[...] * jax.lax.rsqrt(deg)[:, None]


def _scale(degp, xp):
    blk = 2048
    return pl.pallas_call(
        _scale_body,
        grid=(NPAD // blk,),
        in_specs=[
            pl.BlockSpec((2, blk), lambda i: (0, i)),
            pl.BlockSpec((blk, D), lambda i: (i, 0)),
        ],
        out_specs=pl.BlockSpec((blk, D), lambda i: (i, 0)),
        out_shape=jax.ShapeDtypeStruct((NPAD, D), jnp.float32),
    )(degp, xp)


def _dense_body(degp, pp, y, h, Wz, Wr, Wh, Wlz, Wlr, Wlh, Wlin,
                bz, br, bh, blz, blr, blh, blin, z_out, h_out):
    deg = degp[0] + degp[1] + 1.0
    dinv = jax.lax.rsqrt(deg)[:, None]
    A = dinv * (pp[0] + pp[1] + y[...])
    hh = h[...]

    def mm(a, b):
        return jnp.dot(a, b, preferred_element_type=jnp.float32)

    wlz = Wlz[...]
    wlr = Wlr[...]
    wlh = Wlh[...]
    convz = mm(A, Wz[...]) + bz[...]
    Z = jax.nn.sigmoid(mm(convz, wlz[:D]) + mm(hh, wlz[D:]) + blz[...])
    convr = mm(A, Wr[...]) + br[...]
    Rg = jax.nn.sigmoid(mm(convr, wlr[:D]) + mm(hh, wlr[D:]) + blr[...])
    convh = mm(A, Wh[...]) + bh[...]
    Ht = jnp.tanh(mm(convh, wlh[:D]) + mm(hh * Rg, wlh[D:]) + blh[...])
    H = Z * hh + (1.0 - Z) * Ht
    h_out[...] = H
    z_out[...] = mm(jax.nn.relu(H), Wlin[...]) + blin[...]


def _dense(degp, pparts, y, hp, W_z, W_r, W_h, Wl_z, Wl_r, Wl_h, W_lin,
           b_z, b_r, b_h, bl_z, bl_r, bl_h, b_lin):
    blk = 512
    full = lambda shape: pl.BlockSpec(shape, lambda i: tuple(0 for _ in shape))
    return pl.pallas_call(
        _dense_body,
        grid=(NPAD // blk,),
        in_specs=[
            pl.BlockSpec((2, blk), lambda i: (0, i)),
            pl.BlockSpec((2, blk, D), lambda i: (0, i, 0)),
            pl.BlockSpec((blk, D), lambda i: (i, 0)),
            pl.BlockSpec((blk, D), lambda i: (i, 0)),
            full((D, D)), full((D, D)), full((D, D)),
            full((2 * D, D)), full((2 * D, D)), full((2 * D, D)),
            full((D, D)),
            full((1, D)), full((1, D)), full((1, D)),
            full((1, D)), full((1, D)), full((1, D)), full((1, D)),
        ],
        out_specs=[
            pl.BlockSpec((blk, D), lambda i: (i, 0)),
            pl.BlockSpec((blk, D), lambda i: (i, 0)),
        ],
        out_shape=[
            jax.ShapeDtypeStruct((NPAD, D), jnp.float32),
            jax.ShapeDtypeStruct((NPAD, D), jnp.float32),
        ],
    )(degp, pparts, y, hp, W_z, W_r, W_h, Wl_z, Wl_r, Wl_h, W_lin,
      b_z.reshape(1, D), b_r.reshape(1, D), b_h.reshape(1, D),
      bl_z.reshape(1, D), bl_r.reshape(1, D), bl_h.reshape(1, D),
      b_lin.reshape(1, D))


def kernel(node_feat, src, dst, h, W_z, b_z, Wl_z, bl_z, W_r, b_r, Wl_r, bl_r,
           W_h, b_h, Wl_h, bl_h, W_lin, b_lin):
    pad_e = EPAD - src.shape[0]
    srcp = jnp.concatenate([src, jnp.full((pad_e,), N, jnp.int32)])
    dstp = jnp.concatenate([dst, jnp.full((pad_e,), N, jnp.int32)])
    ef = 16 * NB_F * BLK
    src_f = srcp[:ef].reshape(16, NB_F, BLK)
    src_s = srcp[ef:].reshape(16, NB_S, BLK)
    dst_f = dstp[:ef].reshape(16, NB_F, BLK)
    dst_s = dstp[ef:].reshape(16, NB_S, BLK)
    xp = jnp.zeros((NPAD, D), jnp.float32).at[:N].set(node_feat)
    hp = jnp.zeros((NPAD, D), jnp.float32).at[:N].set(h)

    degp = _deg_kernel(dst_f, dst_s)
    y = _scale(degp, xp)
    pparts = _prop_kernel(y, src_f, dst_f, src_s, dst_s)
    z_pad, H_pad = _dense(degp, pparts, y, hp, W_z, W_r, W_h, Wl_z, Wl_r,
                          Wl_h, W_lin, b_z, b_r, b_h, bl_z, bl_r, bl_h, b_lin)
    return (z_pad[:N], H_pad[:N])
